# baseline (speedup 1.0000x reference)
import dataclasses
import functools

import jax
import jax.numpy as jnp
from jax import lax
from jax.experimental import pallas as pl
from jax.experimental.pallas import tpu as pltpu
from jax.experimental.pallas import tpu_sc as plsc

N = 10000
E = 320000
D = 128
H = 128
C = 10
L = 4
B = 64

NC = 2
NS = 16
NW = NC * NS
CHUNK = 128
NCHUNK = 80
HALF = NCHUNK // 2
E_PAD = NW * NCHUNK * CHUNK
PAD = E_PAD - E
JUNK = 64
N_ACC = N + JUNK
RB = 2000
ROW_SPLIT = 632
ROW_LAST = N - (NS - 1) * ROW_SPLIT
DEG_W = 128


def _each_row_slice(s, fn):
    @pl.when(s < NS - 1)
    def _():
        fn(s * ROW_SPLIT, ROW_SPLIT)

    @pl.when(s == NS - 1)
    def _():
        fn((NS - 1) * ROW_SPLIT, ROW_LAST)

_vector_mesh = functools.partial(
    plsc.VectorSubcoreMesh, core_axis_name="c", subcore_axis_name="s"
)


def _sc_degree(dst3, zeros_deg):
    cp = pltpu.CompilerParams()
    if "needs_layout_passes" in pltpu.CompilerParams.__dataclass_fields__:
        cp = dataclasses.replace(cp, needs_layout_passes=False)

    @functools.partial(
        pl.kernel,
        out_type=jax.ShapeDtypeStruct((NW, 1, N), jnp.float32),
        mesh=_vector_mesh(),
        compiler_params=cp,
        scratch_types=[
            pltpu.VMEM((NCHUNK, CHUNK), jnp.int32),
            pltpu.VMEM((N_ACC,), jnp.float32),
            pltpu.SemaphoreType.DMA,
        ],
    )
    def deg_kernel(dst_hbm, z_hbm, out_hbm, dblk, table, sem):
        c = lax.axis_index("c")
        s = lax.axis_index("s")
        wid = c * NS + s
        pltpu.sync_copy(z_hbm, table)
        pltpu.sync_copy(dst_hbm.at[wid], dblk)
        ones16 = jnp.ones((16,), jnp.float32)

        @pl.loop(0, NCHUNK)
        def _(i):
            for k in range(CHUNK // 16):
                idx16 = dblk[i, pl.ds(k * 16, 16)]
                plsc.addupdate_scatter(table, [idx16], ones16)

        pltpu.sync_copy(table.at[pl.ds(0, N)], out_hbm.at[wid, 0])

    return deg_kernel(dst3, zeros_deg)


def _sc_scatter(mprime, src3, dst3, zeros_nh):
    @functools.partial(
        pl.kernel,
        out_type=jax.ShapeDtypeStruct((NC, N, H), jnp.float32),
        mesh=_vector_mesh(),
        scratch_types=[
            pltpu.VMEM((HALF, CHUNK), jnp.int32),
            pltpu.VMEM((HALF, CHUNK), jnp.int32),
            pltpu.VMEM((CHUNK, H), jnp.float32),
            pltpu.VMEM((CHUNK, H), jnp.float32),
            pltpu.VMEM_SHARED((N_ACC, H), jnp.float32),
            pltpu.SemaphoreType.DMA,
            pltpu.SemaphoreType.DMA,
        ],
    )
    def scatter_kernel(m_hbm, src_hbm, dst_hbm, z_hbm, out_hbm,
                       sblk, dblk, rows0, rows1, acc, semA, semB):
        c = lax.axis_index("c")
        s = lax.axis_index("s")
        wid = c * NS + s
        _each_row_slice(s, lambda r0, nr: pltpu.sync_copy(
            z_hbm.at[pl.ds(r0, nr)], acc.at[pl.ds(r0, nr)]))
        plsc.subcore_barrier()

        def gather(i, rows, sem):
            return pltpu.make_async_copy(m_hbm.at[sblk.at[i]], rows, sem)

        for h in range(NCHUNK // HALF):
            pltpu.sync_copy(src_hbm.at[wid, pl.ds(h * HALF, HALF)], sblk)
            pltpu.sync_copy(dst_hbm.at[wid, pl.ds(h * HALF, HALF)], dblk)
            gather(0, rows0, semA).start()

            @pl.loop(0, HALF, step=2)
            def _(i):
                gather(i + 1, rows1, semB).start()
                gather(i, rows0, semA).wait()
                pltpu.sync_copy(rows0, acc.at[dblk.at[i]], add=True)

                @pl.when(i + 2 < HALF)
                def _():
                    gather(i + 2, rows0, semA).start()

                gather(i + 1, rows1, semB).wait()
                pltpu.sync_copy(rows1, acc.at[dblk.at[i + 1]], add=True)

        plsc.subcore_barrier()
        _each_row_slice(s, lambda r0, nr: pltpu.sync_copy(
            acc.at[pl.ds(r0, nr)], out_hbm.at[c].at[pl.ds(r0, nr)]))

    return scatter_kernel(mprime, src3, dst3, zeros_nh)


def _tc_encode(deg_parts, x, W_enc, b_enc2, W1):
    def body(degp_ref, x_ref, we_ref, be_ref, w1_ref, dinv_ref, m_ref):
        deg = 1.0 + jnp.sum(degp_ref[:, 0, :], axis=0)[:, None]
        dinv = lax.rsqrt(deg)
        dinv_ref[...] = dinv
        h0 = jnp.dot(x_ref[...], we_ref[...],
                     preferred_element_type=jnp.float32) + be_ref[...]
        m1 = jnp.dot(h0, w1_ref[...], preferred_element_type=jnp.float32)
        m_ref[...] = m1 * dinv

    return pl.pallas_call(
        body,
        out_shape=(
            jax.ShapeDtypeStruct((N, 1), jnp.float32),
            jax.ShapeDtypeStruct((N, H), jnp.float32),
        ),
    )(deg_parts, x, W_enc, b_enc2, W1)


def _tc_layer(S2, m, dinv, kvec, cvec, W_next):
    def body(s_ref, m_ref, dinv_ref, k_ref, c_ref, w_ref, out_ref):
        dinv = dinv_ref[...]
        h = (s_ref[0] + s_ref[1] + m_ref[...]) * dinv * k_ref[...] + c_ref[...]
        h = jnp.maximum(h, 0.0)
        out_ref[...] = jnp.dot(
            h, w_ref[...], preferred_element_type=jnp.float32) * dinv

    return pl.pallas_call(
        body,
        grid=(N // RB,),
        in_specs=[
            pl.BlockSpec((NC, RB, H), lambda i: (0, i, 0)),
            pl.BlockSpec((RB, H), lambda i: (i, 0)),
            pl.BlockSpec((RB, 1), lambda i: (i, 0)),
            pl.BlockSpec((1, H), lambda i: (0, 0)),
            pl.BlockSpec((1, H), lambda i: (0, 0)),
            pl.BlockSpec((H, H), lambda i: (0, 0)),
        ],
        out_specs=pl.BlockSpec((RB, H), lambda i: (i, 0)),
        out_shape=jax.ShapeDtypeStruct((N, H), jnp.float32),
    )(S2, m, dinv, kvec, cvec, W_next)


def _tc_final(S2, m, dinv, kvec, cvec, batch2, Wc, bc2):
    def body(s_ref, m_ref, dinv_ref, k_ref, c_ref, b_ref, wc_ref, bc_ref,
             out_ref):
        dinv = dinv_ref[...]
        h = (s_ref[0] + s_ref[1] + m_ref[...]) * dinv * k_ref[...] + c_ref[...]
        h = jnp.maximum(h, 0.0)
        seg = lax.broadcasted_iota(jnp.int32, (B, N), 0)
        onehot = jnp.where(seg == b_ref[...], 1.0, 0.0)
        sums = jnp.dot(onehot, h, preferred_element_type=jnp.float32)
        counts = jnp.sum(onehot, axis=1, keepdims=True)
        g = sums / jnp.maximum(counts, 1.0)
        out_ref[...] = jnp.dot(
            g, wc_ref[...], preferred_element_type=jnp.float32) + bc_ref[...]

    return pl.pallas_call(
        body,
        out_shape=jax.ShapeDtypeStruct((B, C), jnp.float32),
    )(S2, m, dinv, kvec, cvec, batch2, Wc, bc2)


def kernel(x, edge_index, batch, W_enc, b_enc, W_stack, b_stack, gamma, beta,
           Wc, bc):
    src = edge_index[0]
    dst = edge_index[1]
    pad = jnp.arange(PAD, dtype=jnp.int32)
    src3 = jnp.concatenate([src, pad % N]).reshape(NW, NCHUNK, CHUNK)
    dst3 = jnp.concatenate([dst, N + pad % JUNK]).reshape(NW, NCHUNK, CHUNK)
    zeros_nh = jnp.zeros((N, H), jnp.float32)
    zeros_deg = jnp.zeros((N_ACC,), jnp.float32)

    bn_scale = lax.rsqrt(jnp.float32(1.0) + jnp.float32(1e-5))
    kmat = bn_scale * gamma
    cmat = b_stack * bn_scale * gamma + beta

    deg_parts = _sc_degree(dst3, zeros_deg)
    dinv, m = _tc_encode(deg_parts, x, W_enc, b_enc.reshape(1, H), W_stack[0])
    for l in range(L):
        S2 = _sc_scatter(m, src3, dst3, zeros_nh)
        kvec = kmat[l].reshape(1, H)
        cvec = cmat[l].reshape(1, H)
        if l < L - 1:
            m = _tc_layer(S2, m, dinv, kvec, cvec, W_stack[l + 1])
        else:
            out = _tc_final(S2, m, dinv, kvec, cvec,
                            batch.reshape(1, N), Wc, bc.reshape(1, C))
    return out


# speedup vs baseline: 1.0051x; 1.0051x over previous
import dataclasses
import functools

import jax
import jax.numpy as jnp
from jax import lax
from jax.experimental import pallas as pl
from jax.experimental.pallas import tpu as pltpu
from jax.experimental.pallas import tpu_sc as plsc

N = 10000
E = 320000
D = 128
H = 128
C = 10
L = 4
B = 64

NC = 2
NS = 16
NW = NC * NS
CHUNK = 128
NCHUNK = 80
HALF = NCHUNK // 2
E_PAD = NW * NCHUNK * CHUNK
PAD = E_PAD - E
JUNK = 64
N_ACC = N + JUNK
RB = 2000
ROW_SPLIT = 632
ROW_LAST = N - (NS - 1) * ROW_SPLIT


def _each_row_slice(s, fn):
    @pl.when(s < NS - 1)
    def _():
        fn(s * ROW_SPLIT, ROW_SPLIT)

    @pl.when(s == NS - 1)
    def _():
        fn((NS - 1) * ROW_SPLIT, ROW_LAST)

_vector_mesh = functools.partial(
    plsc.VectorSubcoreMesh, core_axis_name="c", subcore_axis_name="s"
)


def _sc_degree(dst3, zeros_deg):
    cp = pltpu.CompilerParams()
    if "needs_layout_passes" in pltpu.CompilerParams.__dataclass_fields__:
        cp = dataclasses.replace(cp, needs_layout_passes=False)

    @functools.partial(
        pl.kernel,
        out_type=jax.ShapeDtypeStruct((NW, 1, N), jnp.float32),
        mesh=_vector_mesh(),
        compiler_params=cp,
        scratch_types=[
            pltpu.VMEM((NCHUNK, CHUNK), jnp.int32),
            pltpu.VMEM((N_ACC,), jnp.float32),
            pltpu.SemaphoreType.DMA,
        ],
    )
    def deg_kernel(dst_hbm, z_hbm, out_hbm, dblk, table, sem):
        c = lax.axis_index("c")
        s = lax.axis_index("s")
        wid = c * NS + s
        pltpu.sync_copy(z_hbm, table)
        pltpu.sync_copy(dst_hbm.at[wid], dblk)
        ones16 = jnp.ones((16,), jnp.float32)

        @pl.loop(0, NCHUNK)
        def _(i):
            for k in range(CHUNK // 16):
                idx16 = dblk[i, pl.ds(k * 16, 16)]
                plsc.addupdate_scatter(table, [idx16], ones16)

        pltpu.sync_copy(table.at[pl.ds(0, N)], out_hbm.at[wid, 0])

    return deg_kernel(dst3, zeros_deg)


def _sc_scatter(mprime, src3, dst3, zeros_nh):
    @functools.partial(
        pl.kernel,
        out_type=jax.ShapeDtypeStruct((NC, N, H), jnp.float32),
        mesh=_vector_mesh(),
        scratch_types=[
            pltpu.VMEM((HALF, CHUNK), jnp.int32),
            pltpu.VMEM((HALF, CHUNK), jnp.int32),
            pltpu.VMEM((CHUNK, H), jnp.float32),
            pltpu.VMEM((CHUNK, H), jnp.float32),
            pltpu.VMEM_SHARED((N_ACC, H), jnp.float32),
            pltpu.SemaphoreType.DMA,
            pltpu.SemaphoreType.DMA,
        ],
    )
    def scatter_kernel(m_hbm, src_hbm, dst_hbm, z_hbm, out_hbm,
                       sblk, dblk, rows0, rows1, acc, semA, semB):
        c = lax.axis_index("c")
        s = lax.axis_index("s")
        wid = c * NS + s
        _each_row_slice(s, lambda r0, nr: pltpu.sync_copy(
            z_hbm.at[pl.ds(r0, nr)], acc.at[pl.ds(r0, nr)]))
        plsc.subcore_barrier()

        def gather(i, rows, sem):
            return pltpu.make_async_copy(m_hbm.at[sblk.at[i]], rows, sem)

        for h in range(NCHUNK // HALF):
            pltpu.sync_copy(src_hbm.at[wid, pl.ds(h * HALF, HALF)], sblk)
            pltpu.sync_copy(dst_hbm.at[wid, pl.ds(h * HALF, HALF)], dblk)
            gather(0, rows0, semA).start()

            @pl.loop(0, HALF, step=2)
            def _(i):
                gather(i + 1, rows1, semB).start()
                gather(i, rows0, semA).wait()
                pltpu.sync_copy(rows0, acc.at[dblk.at[i]], add=True)

                @pl.when(i + 2 < HALF)
                def _():
                    gather(i + 2, rows0, semA).start()

                gather(i + 1, rows1, semB).wait()
                pltpu.sync_copy(rows1, acc.at[dblk.at[i + 1]], add=True)

        plsc.subcore_barrier()
        _each_row_slice(s, lambda r0, nr: pltpu.sync_copy(
            acc.at[pl.ds(r0, nr)], out_hbm.at[c].at[pl.ds(r0, nr)]))

    return scatter_kernel(mprime, src3, dst3, zeros_nh)


def _tc_encode(deg_parts, x, W_enc, b_enc2, W1):
    def body(degp_ref, x_ref, we_ref, be_ref, w1_ref, dinv_ref, m_ref):
        deg = 1.0 + jnp.sum(degp_ref[:, 0, :], axis=0)[:, None]
        dinv = lax.rsqrt(deg)
        dinv_ref[...] = dinv
        h0 = jnp.dot(x_ref[...], we_ref[...],
                     preferred_element_type=jnp.float32) + be_ref[...]
        m1 = jnp.dot(h0, w1_ref[...], preferred_element_type=jnp.float32)
        m_ref[...] = m1 * dinv

    return pl.pallas_call(
        body,
        out_shape=(
            jax.ShapeDtypeStruct((N, 1), jnp.float32),
            jax.ShapeDtypeStruct((N, H), jnp.float32),
        ),
    )(deg_parts, x, W_enc, b_enc2, W1)


def _tc_layer(S2, m, dinv, kvec, cvec, W_next):
    def body(s_ref, m_ref, dinv_ref, k_ref, c_ref, w_ref, out_ref):
        dinv = dinv_ref[...]
        h = (s_ref[0] + s_ref[1] + m_ref[...]) * dinv * k_ref[...] + c_ref[...]
        h = jnp.maximum(h, 0.0)
        out_ref[...] = jnp.dot(
            h, w_ref[...], preferred_element_type=jnp.float32) * dinv

    return pl.pallas_call(
        body,
        grid=(N // RB,),
        in_specs=[
            pl.BlockSpec((NC, RB, H), lambda i: (0, i, 0)),
            pl.BlockSpec((RB, H), lambda i: (i, 0)),
            pl.BlockSpec((RB, 1), lambda i: (i, 0)),
            pl.BlockSpec((1, H), lambda i: (0, 0)),
            pl.BlockSpec((1, H), lambda i: (0, 0)),
            pl.BlockSpec((H, H), lambda i: (0, 0)),
        ],
        out_specs=pl.BlockSpec((RB, H), lambda i: (i, 0)),
        out_shape=jax.ShapeDtypeStruct((N, H), jnp.float32),
    )(S2, m, dinv, kvec, cvec, W_next)


def _tc_final(S2, m, dinv, kvec, cvec, batch2, Wc, bc2):
    def body(s_ref, m_ref, dinv_ref, k_ref, c_ref, b_ref, wc_ref, bc_ref,
             out_ref):
        dinv = dinv_ref[...]
        h = (s_ref[0] + s_ref[1] + m_ref[...]) * dinv * k_ref[...] + c_ref[...]
        h = jnp.maximum(h, 0.0)
        seg = lax.broadcasted_iota(jnp.int32, (B, N), 0)
        onehot = jnp.where(seg == b_ref[...], 1.0, 0.0)
        sums = jnp.dot(onehot, h, preferred_element_type=jnp.float32)
        counts = jnp.sum(onehot, axis=1, keepdims=True)
        g = sums / jnp.maximum(counts, 1.0)
        out_ref[...] = jnp.dot(
            g, wc_ref[...], preferred_element_type=jnp.float32) + bc_ref[...]

    return pl.pallas_call(
        body,
        out_shape=jax.ShapeDtypeStruct((B, C), jnp.float32),
    )(S2, m, dinv, kvec, cvec, batch2, Wc, bc2)


def kernel(x, edge_index, batch, W_enc, b_enc, W_stack, b_stack, gamma, beta,
           Wc, bc):
    src = edge_index[0]
    dst = edge_index[1]
    pad = jnp.arange(PAD, dtype=jnp.int32)
    src3 = jnp.concatenate([src, pad % N]).reshape(NW, NCHUNK, CHUNK)
    dst3 = jnp.concatenate([dst, N + pad % JUNK]).reshape(NW, NCHUNK, CHUNK)
    zeros_nh = jnp.zeros((N, H), jnp.float32)
    zeros_deg = jnp.zeros((N_ACC,), jnp.float32)

    bn_scale = lax.rsqrt(jnp.float32(1.0) + jnp.float32(1e-5))
    kmat = bn_scale * gamma
    cmat = b_stack * bn_scale * gamma + beta

    deg_parts = _sc_degree(dst3, zeros_deg)
    dinv, m = _tc_encode(deg_parts, x, W_enc, b_enc.reshape(1, H), W_stack[0])
    for l in range(L):
        S2 = _sc_scatter(m, src3, dst3, zeros_nh)
        kvec = kmat[l].reshape(1, H)
        cvec = cmat[l].reshape(1, H)
        if l < L - 1:
            m = _tc_layer(S2, m, dinv, kvec, cvec, W_stack[l + 1])
        else:
            out = _tc_final(S2, m, dinv, kvec, cvec,
                            batch.reshape(1, N), Wc, bc.reshape(1, C))
    return out
